# baseline (speedup 1.0000x reference)
import functools

import jax
import jax.numpy as jnp
from jax import lax
from jax.experimental import pallas as pl
from jax.experimental.pallas import tpu as pltpu
from jax.experimental.pallas import tpu_sc as plsc

Qn = 1024
DIN = 256
DM = 64
Kn = 100000
BK = 2048
NB = (Kn + BK - 1) // BK
BLUR = 0.9


def _tc_body(x_ref, w_ref, b_ref, keys_ref, bi_ref, loss_ref,
             feats_ref, qsq_ref, cols_ref, bm_ref):
    j = pl.program_id(0)

    @pl.when(j == 0)
    def _init():
        f = jax.nn.gelu(
            jnp.dot(x_ref[...], w_ref[...], preferred_element_type=jnp.float32)
            + b_ref[...])
        feats_ref[:, :DM] = f * (-2.0)
        feats_ref[:, DM:] = jnp.ones((Qn, 1), jnp.float32)
        qsq_ref[...] = jnp.sum(f * f, axis=1, keepdims=True)
        cols_ref[...] = lax.broadcasted_iota(
            jnp.int32, (Qn, BK), 1).astype(jnp.float32)
        bm_ref[...] = jnp.full((Qn, 1), jnp.inf, jnp.float32)
        bi_ref[...] = jnp.zeros((Qn, 1), jnp.int32)

    kb = keys_ref[...]
    rows = lax.broadcasted_iota(jnp.int32, (BK, 1), 0) + j * BK
    valid = rows < Kn
    kb = jnp.where(valid, kb, 0.0)
    kk = kb * kb
    ones_col = jnp.ones((DM, 1), jnp.float32)
    ksq_col = (lax.dot_general(kk, ones_col, (((1,), (0,)), ((), ())),
                               precision=lax.Precision.HIGHEST,
                               preferred_element_type=jnp.float32)
               + jnp.where(valid, 0.0, 1e9))
    k_aug = jnp.concatenate([kb, ksq_col], axis=1)
    m = lax.dot_general(feats_ref[...], k_aug, (((1,), (1,)), ((), ())),
                        preferred_element_type=jnp.float32)

    bm_old = bm_ref[...]
    blk_min = jnp.min(m, axis=1, keepdims=True)
    blk_arg = jnp.min(jnp.where(m == blk_min, cols_ref[...], jnp.float32(2**30)),
                      axis=1, keepdims=True).astype(jnp.int32) + j * BK
    upd = blk_min < bm_old
    bm_ref[...] = jnp.where(upd, blk_min, bm_old)
    bi_ref[...] = jnp.where(upd, blk_arg, bi_ref[...])

    @pl.when(j == NB - 1)
    def _fin():
        loss_ref[...] = jnp.sqrt(
            jnp.maximum(qsq_ref[...] + bm_ref[...], 1e-12))


def _tc_search(x, keys, W, b2):
    return pl.pallas_call(
        _tc_body,
        grid=(NB,),
        in_specs=[
            pl.BlockSpec((Qn, DIN), lambda j: (0, 0)),
            pl.BlockSpec((DIN, DM), lambda j: (0, 0)),
            pl.BlockSpec((1, DM), lambda j: (0, 0)),
            pl.BlockSpec((BK, DM), lambda j: (j, 0)),
        ],
        out_specs=[
            pl.BlockSpec((Qn, 1), lambda j: (0, 0)),
            pl.BlockSpec((Qn, 1), lambda j: (0, 0)),
        ],
        out_shape=[
            jax.ShapeDtypeStruct((Qn, 1), jnp.int32),
            jax.ShapeDtypeStruct((Qn, 1), jnp.float32),
        ],
        scratch_shapes=[
            pltpu.VMEM((Qn, DM + 1), jnp.float32),
            pltpu.VMEM((Qn, 1), jnp.float32),
            pltpu.VMEM((Qn, BK), jnp.float32),
            pltpu.VMEM((Qn, 1), jnp.float32),
        ],
        compiler_params=pltpu.CompilerParams(
            dimension_semantics=("arbitrary",)),
    )(x, W, b2, keys)


def _sc_gather(values, idx, loss):
    info = plsc.get_sparse_core_info()
    nw = info.num_cores * info.num_subcores
    bpw = Qn // nw
    mesh = plsc.VectorSubcoreMesh(core_axis_name="c", subcore_axis_name="s")

    @functools.partial(
        pl.kernel, mesh=mesh,
        out_type=jax.ShapeDtypeStruct((Qn,), jnp.float32),
        scratch_types=[
            pltpu.VMEM((bpw,), jnp.int32),
            pltpu.VMEM((bpw,), jnp.float32),
            pltpu.VMEM((bpw,), jnp.float32),
            pltpu.VMEM((bpw,), jnp.float32),
            pltpu.SemaphoreType.DMA,
        ],
    )
    def k(values_hbm, idx_hbm, loss_hbm, out_hbm, idx_v, loss_v, vals_v,
          out_v, sem):
        wid = lax.axis_index("s") * info.num_cores + lax.axis_index("c")
        base = wid * bpw
        pltpu.sync_copy(idx_hbm.at[pl.ds(base, bpw)], idx_v)
        pltpu.sync_copy(loss_hbm.at[pl.ds(base, bpw)], loss_v)
        pltpu.async_copy(values_hbm.at[idx_v], vals_v, sem).wait()
        for t in range(bpw // 16):
            sl = pl.ds(t * 16, 16)
            out_v[sl] = jnp.where(loss_v[sl] <= BLUR, vals_v[sl],
                                  jnp.zeros((16,), jnp.float32))
        pltpu.sync_copy(out_v, out_hbm.at[pl.ds(base, bpw)])

    return k(values, idx, loss)


def kernel(x, keys, values, W, b):
    bi, loss = _tc_search(x, keys, W, b.reshape(1, DM))
    return _sc_gather(values, bi[:, 0], loss[:, 0])


# speedup vs baseline: 1.6151x; 1.6151x over previous
import functools

import jax
import jax.numpy as jnp
from jax import lax
from jax.experimental import pallas as pl
from jax.experimental.pallas import tpu as pltpu
from jax.experimental.pallas import tpu_sc as plsc

Qn = 1024
DIN = 256
DM = 64
Kn = 100000
BK = 2048
NB = (Kn + BK - 1) // BK
BLUR = 0.9


def _tc_body(x_ref, w_ref, b_ref, keys_ref, bi_ref, loss_ref,
             feats_ref, qsq_ref, cols_ref, bm_ref):
    j = pl.program_id(0)

    @pl.when(j == 0)
    def _init():
        f = jax.nn.gelu(
            jnp.dot(x_ref[...], w_ref[...], preferred_element_type=jnp.float32)
            + b_ref[...])
        feats_ref[:, :DM] = f * (-2.0)
        feats_ref[:, DM:] = jnp.ones((Qn, 1), jnp.float32)
        qsq_ref[...] = jnp.sum(f * f, axis=1, keepdims=True)
        cols_ref[...] = lax.broadcasted_iota(
            jnp.int32, (Qn, BK), 1).astype(jnp.float32)
        bm_ref[...] = jnp.full((Qn, 1), jnp.inf, jnp.float32)
        bi_ref[...] = jnp.zeros((Qn, 1), jnp.int32)

    kb = keys_ref[...]
    rows = lax.broadcasted_iota(jnp.int32, (BK, 1), 0) + j * BK
    valid = rows < Kn
    kb = jnp.where(valid, kb, 0.0)
    ksq_col = (jnp.sum(kb * kb, axis=1, keepdims=True)
               + jnp.where(valid, 0.0, 1e9))
    k_aug = jnp.concatenate([kb, ksq_col], axis=1)
    m = lax.dot_general(feats_ref[...], k_aug, (((1,), (1,)), ((), ())),
                        preferred_element_type=jnp.float32)

    bm_old = bm_ref[...]
    blk_min = jnp.min(m, axis=1, keepdims=True)
    blk_arg = jnp.min(jnp.where(m == blk_min, cols_ref[...], jnp.float32(2**30)),
                      axis=1, keepdims=True).astype(jnp.int32) + j * BK
    upd = blk_min < bm_old
    bm_ref[...] = jnp.where(upd, blk_min, bm_old)
    bi_ref[...] = jnp.where(upd, blk_arg, bi_ref[...])

    @pl.when(j == NB - 1)
    def _fin():
        loss_ref[...] = jnp.sqrt(
            jnp.maximum(qsq_ref[...] + bm_ref[...], 1e-12))


def _tc_search(x, keys, W, b2):
    return pl.pallas_call(
        _tc_body,
        grid=(NB,),
        in_specs=[
            pl.BlockSpec((Qn, DIN), lambda j: (0, 0)),
            pl.BlockSpec((DIN, DM), lambda j: (0, 0)),
            pl.BlockSpec((1, DM), lambda j: (0, 0)),
            pl.BlockSpec((BK, DM), lambda j: (j, 0)),
        ],
        out_specs=[
            pl.BlockSpec((Qn, 1), lambda j: (0, 0)),
            pl.BlockSpec((Qn, 1), lambda j: (0, 0)),
        ],
        out_shape=[
            jax.ShapeDtypeStruct((Qn, 1), jnp.int32),
            jax.ShapeDtypeStruct((Qn, 1), jnp.float32),
        ],
        scratch_shapes=[
            pltpu.VMEM((Qn, DM + 1), jnp.float32),
            pltpu.VMEM((Qn, 1), jnp.float32),
            pltpu.VMEM((Qn, BK), jnp.float32),
            pltpu.VMEM((Qn, 1), jnp.float32),
        ],
        compiler_params=pltpu.CompilerParams(
            dimension_semantics=("arbitrary",)),
    )(x, W, b2, keys)


def _sc_gather(values, idx, loss):
    info = plsc.get_sparse_core_info()
    nw = info.num_cores * info.num_subcores
    bpw = Qn // nw
    mesh = plsc.VectorSubcoreMesh(core_axis_name="c", subcore_axis_name="s")

    @functools.partial(
        pl.kernel, mesh=mesh,
        out_type=jax.ShapeDtypeStruct((Qn,), jnp.float32),
        scratch_types=[
            pltpu.VMEM((bpw,), jnp.int32),
            pltpu.VMEM((bpw,), jnp.float32),
            pltpu.VMEM((bpw,), jnp.float32),
            pltpu.VMEM((bpw,), jnp.float32),
            pltpu.SemaphoreType.DMA,
        ],
    )
    def k(values_hbm, idx_hbm, loss_hbm, out_hbm, idx_v, loss_v, vals_v,
          out_v, sem):
        wid = lax.axis_index("s") * info.num_cores + lax.axis_index("c")
        base = wid * bpw
        pltpu.sync_copy(idx_hbm.at[pl.ds(base, bpw)], idx_v)
        pltpu.sync_copy(loss_hbm.at[pl.ds(base, bpw)], loss_v)
        pltpu.async_copy(values_hbm.at[idx_v], vals_v, sem).wait()
        for t in range(bpw // 16):
            sl = pl.ds(t * 16, 16)
            out_v[sl] = jnp.where(loss_v[sl] <= BLUR, vals_v[sl],
                                  jnp.zeros((16,), jnp.float32))
        pltpu.sync_copy(out_v, out_hbm.at[pl.ds(base, bpw)])

    return k(values, idx, loss)


def kernel(x, keys, values, W, b):
    bi, loss = _tc_search(x, keys, W, b.reshape(1, DM))
    return _sc_gather(values, bi[:, 0], loss[:, 0])


# speedup vs baseline: 1.6241x; 1.0056x over previous
import functools

import jax
import jax.numpy as jnp
from jax import lax
from jax.experimental import pallas as pl
from jax.experimental.pallas import tpu as pltpu
from jax.experimental.pallas import tpu_sc as plsc

Qn = 1024
DIN = 256
DM = 64
Kn = 100000
BK = 2048
NB = (Kn + BK - 1) // BK
BLUR = 0.9


def _tc_body(x_ref, w_ref, b_ref, keys_ref, bi_ref, loss_ref,
             feats_ref, qsq_ref, cols_ref, bm_ref):
    j = pl.program_id(0)

    @pl.when(j == 0)
    def _init():
        f = jax.nn.gelu(
            jnp.dot(x_ref[...], w_ref[...], preferred_element_type=jnp.float32)
            + b_ref[...])
        feats_ref[:, :DM] = f * (-2.0)
        feats_ref[:, DM:] = jnp.ones((Qn, 1), jnp.float32)
        qsq_ref[...] = jnp.sum(f * f, axis=1, keepdims=True)
        cols_ref[...] = lax.broadcasted_iota(
            jnp.int32, (Qn, BK), 1).astype(jnp.float32)
        bm_ref[...] = jnp.full((Qn, 1), jnp.inf, jnp.float32)
        bi_ref[...] = jnp.zeros((Qn, 1), jnp.float32)

    kb = keys_ref[...]
    rows = lax.broadcasted_iota(jnp.int32, (BK, 1), 0) + j * BK
    valid = rows < Kn
    kb = jnp.where(valid, kb, 0.0)
    ksq_col = (jnp.sum(kb * kb, axis=1, keepdims=True)
               + jnp.where(valid, 0.0, 1e9))
    k_aug = jnp.concatenate([kb, ksq_col], axis=1)
    m = lax.dot_general(feats_ref[...], k_aug, (((1,), (1,)), ((), ())),
                        preferred_element_type=jnp.float32)

    bm_old = bm_ref[...]
    blk_min = jnp.min(m, axis=1, keepdims=True)
    blk_arg = jnp.min(jnp.where(m == blk_min, cols_ref[...], jnp.float32(2**30)),
                      axis=1, keepdims=True) + jnp.float32(j * BK)
    upd = blk_min < bm_old
    bm_ref[...] = jnp.where(upd, blk_min, bm_old)
    bi_ref[...] = jnp.where(upd, blk_arg, bi_ref[...])

    @pl.when(j == NB - 1)
    def _fin():
        loss_ref[...] = jnp.sqrt(
            jnp.maximum(qsq_ref[...] + bm_ref[...], 1e-12))


def _tc_search(x, keys, W, b2):
    return pl.pallas_call(
        _tc_body,
        grid=(NB,),
        in_specs=[
            pl.BlockSpec((Qn, DIN), lambda j: (0, 0)),
            pl.BlockSpec((DIN, DM), lambda j: (0, 0)),
            pl.BlockSpec((1, DM), lambda j: (0, 0)),
            pl.BlockSpec((BK, DM), lambda j: (j, 0)),
        ],
        out_specs=[
            pl.BlockSpec((Qn, 1), lambda j: (0, 0)),
            pl.BlockSpec((Qn, 1), lambda j: (0, 0)),
        ],
        out_shape=[
            jax.ShapeDtypeStruct((Qn, 1), jnp.float32),
            jax.ShapeDtypeStruct((Qn, 1), jnp.float32),
        ],
        scratch_shapes=[
            pltpu.VMEM((Qn, DM + 1), jnp.float32),
            pltpu.VMEM((Qn, 1), jnp.float32),
            pltpu.VMEM((Qn, BK), jnp.float32),
            pltpu.VMEM((Qn, 1), jnp.float32),
        ],
        compiler_params=pltpu.CompilerParams(
            dimension_semantics=("arbitrary",)),
    )(x, W, b2, keys)


def _sc_gather(values, idx, loss):
    info = plsc.get_sparse_core_info()
    nw = info.num_cores * info.num_subcores
    bpw = Qn // nw
    mesh = plsc.VectorSubcoreMesh(core_axis_name="c", subcore_axis_name="s")

    @functools.partial(
        pl.kernel, mesh=mesh,
        out_type=jax.ShapeDtypeStruct((Qn,), jnp.float32),
        scratch_types=[
            pltpu.VMEM((bpw,), jnp.int32),
            pltpu.VMEM((bpw,), jnp.float32),
            pltpu.VMEM((bpw,), jnp.float32),
            pltpu.VMEM((bpw,), jnp.float32),
            pltpu.SemaphoreType.DMA,
        ],
    )
    def k(values_hbm, idx_hbm, loss_hbm, out_hbm, idx_v, loss_v, vals_v,
          out_v, sem):
        wid = lax.axis_index("s") * info.num_cores + lax.axis_index("c")
        base = wid * bpw
        pltpu.sync_copy(idx_hbm.at[pl.ds(base, bpw)], idx_v)
        pltpu.sync_copy(loss_hbm.at[pl.ds(base, bpw)], loss_v)
        pltpu.async_copy(values_hbm.at[idx_v], vals_v, sem).wait()
        for t in range(bpw // 16):
            sl = pl.ds(t * 16, 16)
            out_v[sl] = jnp.where(loss_v[sl] <= BLUR, vals_v[sl],
                                  jnp.zeros((16,), jnp.float32))
        pltpu.sync_copy(out_v, out_hbm.at[pl.ds(base, bpw)])

    return k(values, idx, loss)


def kernel(x, keys, values, W, b):
    bi, loss = _tc_search(x, keys, W, b.reshape(1, DM))
    return _sc_gather(values, bi[:, 0].astype(jnp.int32), loss[:, 0])


# speedup vs baseline: 2.0590x; 1.2678x over previous
import functools

import jax
import jax.numpy as jnp
from jax import lax
from jax.experimental import pallas as pl
from jax.experimental.pallas import tpu as pltpu
from jax.experimental.pallas import tpu_sc as plsc

Qn = 1024
DIN = 256
DM = 64
Kn = 100000
BK = 2048
NB = (Kn + BK - 1) // BK
BLUR = 0.9


def _tc_body(x_ref, w_ref, b_ref, keys_ref, bi_ref, loss_ref,
             feats_ref, qsq_ref, cols_ref, bm_ref, bj_ref):
    j = pl.program_id(0)

    @pl.when(j == 0)
    def _init():
        f = jax.nn.gelu(
            jnp.dot(x_ref[...], w_ref[...], preferred_element_type=jnp.float32)
            + b_ref[...])
        feats_ref[:, :DM] = f * (-2.0)
        feats_ref[:, DM:] = jnp.ones((Qn, 1), jnp.float32)
        qsq_ref[...] = jnp.sum(f * f, axis=1, keepdims=True)
        cols_ref[...] = lax.broadcasted_iota(jnp.int32, (Qn, BK), 1)
        bm_ref[...] = jnp.full((Qn, 1), jnp.inf, jnp.float32)
        bj_ref[...] = jnp.zeros((Qn, 1), jnp.float32)

    kb = keys_ref[...]
    rows = lax.broadcasted_iota(jnp.int32, (BK, 1), 0) + j * BK
    valid = rows < Kn
    kb = jnp.where(valid, kb, 0.0)
    ksq_col = (jnp.sum(kb * kb, axis=1, keepdims=True)
               + jnp.where(valid, 0.0, 1e9))
    k_aug = jnp.concatenate([kb, ksq_col], axis=1)
    m = lax.dot_general(feats_ref[...], k_aug, (((1,), (1,)), ((), ())),
                        preferred_element_type=jnp.float32)

    z = lax.bitcast_convert_type(
        lax.bitcast_convert_type(m, jnp.int32) | cols_ref[...], jnp.float32)
    zmin = jnp.min(z, axis=1, keepdims=True)
    bm_old = bm_ref[...]
    upd = zmin < bm_old
    bm_ref[...] = jnp.where(upd, zmin, bm_old)
    bj_ref[...] = jnp.where(upd, jnp.float32(j), bj_ref[...])

    @pl.when(j == NB - 1)
    def _fin():
        zi = lax.bitcast_convert_type(bm_ref[...], jnp.int32)
        col = (zi & 2047).astype(jnp.float32)
        bi_ref[...] = bj_ref[...] * jnp.float32(BK) + col
        bm_val = lax.bitcast_convert_type(zi & ~2047, jnp.float32)
        loss_ref[...] = jnp.sqrt(
            jnp.maximum(qsq_ref[...] + bm_val, 1e-12))


def _tc_search(x, keys, W, b2):
    return pl.pallas_call(
        _tc_body,
        grid=(NB,),
        in_specs=[
            pl.BlockSpec((Qn, DIN), lambda j: (0, 0)),
            pl.BlockSpec((DIN, DM), lambda j: (0, 0)),
            pl.BlockSpec((1, DM), lambda j: (0, 0)),
            pl.BlockSpec((BK, DM), lambda j: (j, 0)),
        ],
        out_specs=[
            pl.BlockSpec((Qn, 1), lambda j: (0, 0)),
            pl.BlockSpec((Qn, 1), lambda j: (0, 0)),
        ],
        out_shape=[
            jax.ShapeDtypeStruct((Qn, 1), jnp.float32),
            jax.ShapeDtypeStruct((Qn, 1), jnp.float32),
        ],
        scratch_shapes=[
            pltpu.VMEM((Qn, DM + 1), jnp.float32),
            pltpu.VMEM((Qn, 1), jnp.float32),
            pltpu.VMEM((Qn, BK), jnp.int32),
            pltpu.VMEM((Qn, 1), jnp.float32),
            pltpu.VMEM((Qn, 1), jnp.float32),
        ],
        compiler_params=pltpu.CompilerParams(
            dimension_semantics=("arbitrary",)),
    )(x, W, b2, keys)


def _sc_gather(values, idx, loss):
    info = plsc.get_sparse_core_info()
    nw = info.num_cores * info.num_subcores
    bpw = Qn // nw
    mesh = plsc.VectorSubcoreMesh(core_axis_name="c", subcore_axis_name="s")

    @functools.partial(
        pl.kernel, mesh=mesh,
        out_type=jax.ShapeDtypeStruct((Qn,), jnp.float32),
        scratch_types=[
            pltpu.VMEM((bpw,), jnp.int32),
            pltpu.VMEM((bpw,), jnp.float32),
            pltpu.VMEM((bpw,), jnp.float32),
            pltpu.VMEM((bpw,), jnp.float32),
            pltpu.SemaphoreType.DMA,
        ],
    )
    def k(values_hbm, idx_hbm, loss_hbm, out_hbm, idx_v, loss_v, vals_v,
          out_v, sem):
        wid = lax.axis_index("s") * info.num_cores + lax.axis_index("c")
        base = wid * bpw
        pltpu.sync_copy(idx_hbm.at[pl.ds(base, bpw)], idx_v)
        pltpu.sync_copy(loss_hbm.at[pl.ds(base, bpw)], loss_v)
        pltpu.async_copy(values_hbm.at[idx_v], vals_v, sem).wait()
        for t in range(bpw // 16):
            sl = pl.ds(t * 16, 16)
            out_v[sl] = jnp.where(loss_v[sl] <= BLUR, vals_v[sl],
                                  jnp.zeros((16,), jnp.float32))
        pltpu.sync_copy(out_v, out_hbm.at[pl.ds(base, bpw)])

    return k(values, idx, loss)


def kernel(x, keys, values, W, b):
    bi, loss = _tc_search(x, keys, W, b.reshape(1, DM))
    return _sc_gather(values, bi[:, 0].astype(jnp.int32), loss[:, 0])
